# baseline (speedup 1.0000x reference)
import functools
import math

import jax
import jax.numpy as jnp
from jax import lax
from jax.experimental import pallas as pl
from jax.experimental.pallas import tpu as pltpu
from jax.experimental.pallas import tpu_sc as plsc

_BT = 288
_NW = 32

_INTERPRET = False


def _sc_row_gather(table, idx):
    V, C = table.shape
    B = idx.shape[0]
    b_per_w = B // _NW
    ch = min(b_per_w, 64)
    n_ch = b_per_w // ch
    mesh = plsc.VectorSubcoreMesh(core_axis_name="c", subcore_axis_name="s")

    @functools.partial(
        pl.kernel,
        out_type=jax.ShapeDtypeStruct((B, C), jnp.float32),
        mesh=mesh,
        scratch_types=[
            pltpu.VMEM((b_per_w,), jnp.int32),
            pltpu.VMEM((2, ch, C), jnp.float32),
            pltpu.SemaphoreType.DMA,
            pltpu.SemaphoreType.DMA,
        ],
    )
    def k(table_hbm, idx_hbm, out_hbm, idx_v, rows_v, sem0, sem1):
        wid = lax.axis_index("s") * 2 + lax.axis_index("c")
        base = wid * b_per_w
        sems = (sem0, sem1)
        pltpu.sync_copy(idx_hbm.at[pl.ds(base, b_per_w)], idx_v)
        descs = []
        for j in range(n_ch):
            descs.append(pltpu.async_copy(
                table_hbm.at[idx_v.at[pl.ds(j * ch, ch)]],
                rows_v.at[j % 2], sems[j % 2]))
            if j >= 1:
                descs[j - 1].wait()
                pltpu.sync_copy(rows_v.at[(j - 1) % 2],
                                out_hbm.at[pl.ds(base + (j - 1) * ch, ch)])
        descs[n_ch - 1].wait()
        pltpu.sync_copy(rows_v.at[(n_ch - 1) % 2],
                        out_hbm.at[pl.ds(base + (n_ch - 1) * ch, ch)])

    return k(table, idx)


def _sc_row_scatter(src, idx, n_out):
    B, C = src.shape
    b_per_w = B // _NW
    ch = min(b_per_w, 64)
    n_ch = b_per_w // ch
    mesh = plsc.VectorSubcoreMesh(core_axis_name="c", subcore_axis_name="s")

    @functools.partial(
        pl.kernel,
        out_type=jax.ShapeDtypeStruct((n_out, C), jnp.float32),
        mesh=mesh,
        scratch_types=[
            pltpu.VMEM((2, ch), jnp.int32),
            pltpu.VMEM((2, ch, C), jnp.float32),
            pltpu.SemaphoreType.DMA,
            pltpu.SemaphoreType.DMA,
        ],
    )
    def k(src_hbm, idx_hbm, out_hbm, idx_v, rows_v, sem0, sem1):
        wid = lax.axis_index("s") * 2 + lax.axis_index("c")
        base = wid * b_per_w
        sems = (sem0, sem1)
        loads = []
        for j in range(n_ch):
            pltpu.sync_copy(idx_hbm.at[pl.ds(base + j * ch, ch)],
                            idx_v.at[j % 2])
            loads.append(pltpu.async_copy(
                src_hbm.at[pl.ds(base + j * ch, ch)],
                rows_v.at[j % 2], sems[j % 2]))
            if j >= 1:
                loads[j - 1].wait()
                pltpu.sync_copy(rows_v.at[(j - 1) % 2],
                                out_hbm.at[idx_v.at[(j - 1) % 2]])
        loads[n_ch - 1].wait()
        pltpu.sync_copy(rows_v.at[(n_ch - 1) % 2],
                        out_hbm.at[idx_v.at[(n_ch - 1) % 2]])

    return k(src, idx)


def _gelu(h):
    return 0.5 * h * (1.0 + jnp.tanh(
        math.sqrt(2.0 / math.pi) * (h + 0.044715 * (h * h * h))))


def _ffn_block(be_ref, act_ref, x_ref, w1a_ref, w1b_ref, b1_ref,
               w2a_ref, w2b_ref, b2_ref, o_ref):
    b = pl.program_id(0)
    HH = w1a_ref.shape[2]

    @pl.when(b < act_ref[0])
    def _():
        x = x_ref[...]
        ha = _gelu(jnp.dot(x, w1a_ref[0], preferred_element_type=jnp.float32)
                   + b1_ref[0, 0, :HH])
        hb = _gelu(jnp.dot(x, w1b_ref[0], preferred_element_type=jnp.float32)
                   + b1_ref[0, 0, HH:])
        o = jnp.dot(ha, w2a_ref[0], preferred_element_type=jnp.float32)
        o = o + jnp.dot(hb, w2b_ref[0], preferred_element_type=jnp.float32)
        o_ref[...] = o + b2_ref[0, 0]


def kernel(x, Wr, W1, b1, W2, b2):
    Bx, Tx, Cx = x.shape
    E, C, H = W1.shape
    N = Bx * Tx
    NB = N // _BT + E
    PADDED = NB * _BT

    x_flat = x.reshape(N, Cx)

    logits = x_flat @ Wr
    eid = jnp.argmax(logits, axis=1).astype(jnp.int32)

    G = 8
    NG = N // G
    eidg = eid.reshape(G, NG)
    ohg = (eidg[:, :, None] == jnp.arange(E, dtype=jnp.int32)).astype(jnp.int32)
    ohp = ohg.transpose(1, 0, 2)
    cs = jnp.cumsum(ohp.reshape(NG, G * E), axis=0)
    totals = cs[-1].reshape(G, E)
    goff = jnp.concatenate(
        [jnp.zeros((1, E), jnp.int32), jnp.cumsum(totals, axis=0)[:-1]], axis=0)
    counts = jnp.sum(totals, axis=0)
    rank_incl = cs.reshape(NG, G, E) + goff[None]
    rank = jnp.sum(ohp * rank_incl, axis=2)
    rank = rank.T.reshape(N) - 1
    padded = ((counts + _BT - 1) // _BT) * _BT
    bounds = jnp.cumsum(padded)
    starts = bounds - padded
    pos = (starts[eid] + rank).astype(jnp.int32)
    total = bounds[-1]
    nb_active = (total // _BT).astype(jnp.int32)

    bidx = jnp.arange(NB, dtype=jnp.int32) * _BT
    block_expert = jnp.sum((bounds[None, :] <= bidx[:, None]).astype(jnp.int32),
                           axis=1)
    block_expert = jnp.minimum(block_expert, E - 1)
    last_e = block_expert[jnp.maximum(nb_active - 1, 0)]
    block_expert = jnp.where(bidx < total, block_expert, last_e)

    x_sorted = _sc_row_scatter(x_flat, pos, PADDED)

    grid_spec = pltpu.PrefetchScalarGridSpec(
        num_scalar_prefetch=2,
        grid=(NB,),
        in_specs=[
            pl.BlockSpec((_BT, C), lambda b, be, act: (b, 0)),
            pl.BlockSpec((1, C, H // 2), lambda b, be, act: (be[b], 0, 0)),
            pl.BlockSpec((1, C, H // 2), lambda b, be, act: (be[b], 0, 1)),
            pl.BlockSpec((1, 1, H), lambda b, be, act: (be[b], 0, 0)),
            pl.BlockSpec((1, H // 2, C), lambda b, be, act: (be[b], 0, 0)),
            pl.BlockSpec((1, H // 2, C), lambda b, be, act: (be[b], 1, 0)),
            pl.BlockSpec((1, 1, C), lambda b, be, act: (be[b], 0, 0)),
        ],
        out_specs=pl.BlockSpec((_BT, C), lambda b, be, act: (b, 0)),
    )
    out_buf = pl.pallas_call(
        _ffn_block,
        grid_spec=grid_spec,
        out_shape=jax.ShapeDtypeStruct((PADDED, C), jnp.float32),
        interpret=_INTERPRET,
    )(block_expert, jnp.full((1,), nb_active, jnp.int32),
      x_sorted, W1, W1, b1.reshape(E, 1, H), W2, W2, b2.reshape(E, 1, C))

    y_flat = _sc_row_gather(out_buf, pos)
    return y_flat.reshape(Bx, Tx, Cx)


# speedup vs baseline: 1.0341x; 1.0341x over previous
import functools
import math

import jax
import jax.numpy as jnp
from jax import lax
from jax.experimental import pallas as pl
from jax.experimental.pallas import tpu as pltpu
from jax.experimental.pallas import tpu_sc as plsc

_BT = 384
_NW = 32

_INTERPRET = False


def _sc_row_gather(table, idx):
    V, C = table.shape
    B = idx.shape[0]
    b_per_w = B // _NW
    ch = min(b_per_w, 64)
    n_ch = b_per_w // ch
    mesh = plsc.VectorSubcoreMesh(core_axis_name="c", subcore_axis_name="s")

    @functools.partial(
        pl.kernel,
        out_type=jax.ShapeDtypeStruct((B, C), jnp.float32),
        mesh=mesh,
        scratch_types=[
            pltpu.VMEM((b_per_w,), jnp.int32),
            pltpu.VMEM((2, ch, C), jnp.float32),
            pltpu.SemaphoreType.DMA,
            pltpu.SemaphoreType.DMA,
        ],
    )
    def k(table_hbm, idx_hbm, out_hbm, idx_v, rows_v, sem0, sem1):
        wid = lax.axis_index("s") * 2 + lax.axis_index("c")
        base = wid * b_per_w
        sems = (sem0, sem1)
        pltpu.sync_copy(idx_hbm.at[pl.ds(base, b_per_w)], idx_v)
        descs = []
        for j in range(n_ch):
            descs.append(pltpu.async_copy(
                table_hbm.at[idx_v.at[pl.ds(j * ch, ch)]],
                rows_v.at[j % 2], sems[j % 2]))
            if j >= 1:
                descs[j - 1].wait()
                pltpu.sync_copy(rows_v.at[(j - 1) % 2],
                                out_hbm.at[pl.ds(base + (j - 1) * ch, ch)])
        descs[n_ch - 1].wait()
        pltpu.sync_copy(rows_v.at[(n_ch - 1) % 2],
                        out_hbm.at[pl.ds(base + (n_ch - 1) * ch, ch)])

    return k(table, idx)


def _sc_row_scatter(src, idx, n_out):
    B, C = src.shape
    b_per_w = B // _NW
    ch = min(b_per_w, 64)
    n_ch = b_per_w // ch
    mesh = plsc.VectorSubcoreMesh(core_axis_name="c", subcore_axis_name="s")

    @functools.partial(
        pl.kernel,
        out_type=jax.ShapeDtypeStruct((n_out, C), jnp.float32),
        mesh=mesh,
        scratch_types=[
            pltpu.VMEM((2, ch), jnp.int32),
            pltpu.VMEM((2, ch, C), jnp.float32),
            pltpu.SemaphoreType.DMA,
            pltpu.SemaphoreType.DMA,
        ],
    )
    def k(src_hbm, idx_hbm, out_hbm, idx_v, rows_v, sem0, sem1):
        wid = lax.axis_index("s") * 2 + lax.axis_index("c")
        base = wid * b_per_w
        sems = (sem0, sem1)
        loads = []
        for j in range(n_ch):
            pltpu.sync_copy(idx_hbm.at[pl.ds(base + j * ch, ch)],
                            idx_v.at[j % 2])
            loads.append(pltpu.async_copy(
                src_hbm.at[pl.ds(base + j * ch, ch)],
                rows_v.at[j % 2], sems[j % 2]))
            if j >= 1:
                loads[j - 1].wait()
                pltpu.sync_copy(rows_v.at[(j - 1) % 2],
                                out_hbm.at[idx_v.at[(j - 1) % 2]])
        loads[n_ch - 1].wait()
        pltpu.sync_copy(rows_v.at[(n_ch - 1) % 2],
                        out_hbm.at[idx_v.at[(n_ch - 1) % 2]])

    return k(src, idx)


def _gelu(h):
    return 0.5 * h * (1.0 + jnp.tanh(
        math.sqrt(2.0 / math.pi) * (h + 0.044715 * (h * h * h))))


def _ffn_block(be_ref, act_ref, x_ref, w1a_ref, w1b_ref, b1_ref,
               w2a_ref, w2b_ref, b2_ref, o_ref):
    b = pl.program_id(0)
    HH = w1a_ref.shape[2]

    @pl.when(b < act_ref[0])
    def _():
        x = x_ref[...]
        ha = _gelu(jnp.dot(x, w1a_ref[0], preferred_element_type=jnp.float32)
                   + b1_ref[0, 0, :HH])
        hb = _gelu(jnp.dot(x, w1b_ref[0], preferred_element_type=jnp.float32)
                   + b1_ref[0, 0, HH:])
        o = jnp.dot(ha, w2a_ref[0], preferred_element_type=jnp.float32)
        o = o + jnp.dot(hb, w2b_ref[0], preferred_element_type=jnp.float32)
        o_ref[...] = o + b2_ref[0, 0]


def kernel(x, Wr, W1, b1, W2, b2):
    Bx, Tx, Cx = x.shape
    E, C, H = W1.shape
    N = Bx * Tx
    NB = N // _BT + E
    PADDED = NB * _BT

    x_flat = x.reshape(N, Cx)

    logits = x_flat @ Wr
    eid = jnp.argmax(logits, axis=1).astype(jnp.int32)

    G = 8
    NG = N // G
    eidg = eid.reshape(G, NG)
    ohg = (eidg[:, :, None] == jnp.arange(E, dtype=jnp.int32)).astype(jnp.int32)
    ohp = ohg.transpose(1, 0, 2)
    cs = jnp.cumsum(ohp.reshape(NG, G * E), axis=0)
    totals = cs[-1].reshape(G, E)
    goff = jnp.concatenate(
        [jnp.zeros((1, E), jnp.int32), jnp.cumsum(totals, axis=0)[:-1]], axis=0)
    counts = jnp.sum(totals, axis=0)
    rank_incl = cs.reshape(NG, G, E) + goff[None]
    rank = jnp.sum(ohp * rank_incl, axis=2)
    rank = rank.T.reshape(N) - 1
    padded = ((counts + _BT - 1) // _BT) * _BT
    bounds = jnp.cumsum(padded)
    starts = bounds - padded
    pos = (starts[eid] + rank).astype(jnp.int32)
    total = bounds[-1]
    nb_active = (total // _BT).astype(jnp.int32)

    bidx = jnp.arange(NB, dtype=jnp.int32) * _BT
    block_expert = jnp.sum((bounds[None, :] <= bidx[:, None]).astype(jnp.int32),
                           axis=1)
    block_expert = jnp.minimum(block_expert, E - 1)
    last_e = block_expert[jnp.maximum(nb_active - 1, 0)]
    block_expert = jnp.where(bidx < total, block_expert, last_e)

    x_sorted = _sc_row_scatter(x_flat, pos, PADDED)

    grid_spec = pltpu.PrefetchScalarGridSpec(
        num_scalar_prefetch=2,
        grid=(NB,),
        in_specs=[
            pl.BlockSpec((_BT, C), lambda b, be, act: (b, 0)),
            pl.BlockSpec((1, C, H // 2), lambda b, be, act: (be[b], 0, 0)),
            pl.BlockSpec((1, C, H // 2), lambda b, be, act: (be[b], 0, 1)),
            pl.BlockSpec((1, 1, H), lambda b, be, act: (be[b], 0, 0)),
            pl.BlockSpec((1, H // 2, C), lambda b, be, act: (be[b], 0, 0)),
            pl.BlockSpec((1, H // 2, C), lambda b, be, act: (be[b], 1, 0)),
            pl.BlockSpec((1, 1, C), lambda b, be, act: (be[b], 0, 0)),
        ],
        out_specs=pl.BlockSpec((_BT, C), lambda b, be, act: (b, 0)),
    )
    out_buf = pl.pallas_call(
        _ffn_block,
        grid_spec=grid_spec,
        out_shape=jax.ShapeDtypeStruct((PADDED, C), jnp.float32),
        interpret=_INTERPRET,
    )(block_expert, jnp.full((1,), nb_active, jnp.int32),
      x_sorted, W1, W1, b1.reshape(E, 1, H), W2, W2, b2.reshape(E, 1, C))

    y_flat = _sc_row_gather(out_buf, pos)
    return y_flat.reshape(Bx, Tx, Cx)


# speedup vs baseline: 1.0512x; 1.0165x over previous
import functools
import math

import jax
import jax.numpy as jnp
from jax import lax
from jax.experimental import pallas as pl
from jax.experimental.pallas import tpu as pltpu
from jax.experimental.pallas import tpu_sc as plsc

_BT = 320
_NW = 32


def _sc_row_gather(table, idx):
    V, C = table.shape
    B = idx.shape[0]
    b_per_w = B // _NW
    ch = min(b_per_w, 64)
    n_ch = b_per_w // ch
    mesh = plsc.VectorSubcoreMesh(core_axis_name="c", subcore_axis_name="s")

    @functools.partial(
        pl.kernel,
        out_type=jax.ShapeDtypeStruct((B, C), jnp.float32),
        mesh=mesh,
        scratch_types=[
            pltpu.VMEM((b_per_w,), jnp.int32),
            pltpu.VMEM((2, ch, C), jnp.float32),
            pltpu.SemaphoreType.DMA,
            pltpu.SemaphoreType.DMA,
        ],
    )
    def k(table_hbm, idx_hbm, out_hbm, idx_v, rows_v, sem0, sem1):
        wid = lax.axis_index("s") * 2 + lax.axis_index("c")
        base = wid * b_per_w
        sems = (sem0, sem1)
        pltpu.sync_copy(idx_hbm.at[pl.ds(base, b_per_w)], idx_v)
        descs = []
        for j in range(n_ch):
            descs.append(pltpu.async_copy(
                table_hbm.at[idx_v.at[pl.ds(j * ch, ch)]],
                rows_v.at[j % 2], sems[j % 2]))
            if j >= 1:
                descs[j - 1].wait()
                pltpu.sync_copy(rows_v.at[(j - 1) % 2],
                                out_hbm.at[pl.ds(base + (j - 1) * ch, ch)])
        descs[n_ch - 1].wait()
        pltpu.sync_copy(rows_v.at[(n_ch - 1) % 2],
                        out_hbm.at[pl.ds(base + (n_ch - 1) * ch, ch)])

    return k(table, idx)


def _sc_row_scatter(src, idx, n_out):
    B, C = src.shape
    b_per_w = B // _NW
    ch = min(b_per_w, 64)
    n_ch = b_per_w // ch
    mesh = plsc.VectorSubcoreMesh(core_axis_name="c", subcore_axis_name="s")

    @functools.partial(
        pl.kernel,
        out_type=jax.ShapeDtypeStruct((n_out, C), jnp.float32),
        mesh=mesh,
        scratch_types=[
            pltpu.VMEM((2, ch), jnp.int32),
            pltpu.VMEM((2, ch, C), jnp.float32),
            pltpu.SemaphoreType.DMA,
            pltpu.SemaphoreType.DMA,
        ],
    )
    def k(src_hbm, idx_hbm, out_hbm, idx_v, rows_v, sem0, sem1):
        wid = lax.axis_index("s") * 2 + lax.axis_index("c")
        base = wid * b_per_w
        sems = (sem0, sem1)
        loads = []
        for j in range(n_ch):
            pltpu.sync_copy(idx_hbm.at[pl.ds(base + j * ch, ch)],
                            idx_v.at[j % 2])
            loads.append(pltpu.async_copy(
                src_hbm.at[pl.ds(base + j * ch, ch)],
                rows_v.at[j % 2], sems[j % 2]))
            if j >= 1:
                loads[j - 1].wait()
                pltpu.sync_copy(rows_v.at[(j - 1) % 2],
                                out_hbm.at[idx_v.at[(j - 1) % 2]])
        loads[n_ch - 1].wait()
        pltpu.sync_copy(rows_v.at[(n_ch - 1) % 2],
                        out_hbm.at[idx_v.at[(n_ch - 1) % 2]])

    return k(src, idx)


def _gelu(h):
    return 0.5 * h * (1.0 + jnp.tanh(
        math.sqrt(2.0 / math.pi) * (h + 0.044715 * (h * h * h))))


def _ffn_block(be_ref, act_ref, x_ref, w1a_ref, w1b_ref, b1_ref,
               w2a_ref, w2b_ref, b2_ref, o_ref):
    b = pl.program_id(0)
    HH = w1a_ref.shape[2]

    @pl.when(b < act_ref[0])
    def _():
        x = x_ref[...]
        ha = _gelu(jnp.dot(x, w1a_ref[0], preferred_element_type=jnp.float32)
                   + b1_ref[0, 0, :HH])
        hb = _gelu(jnp.dot(x, w1b_ref[0], preferred_element_type=jnp.float32)
                   + b1_ref[0, 0, HH:])
        o = jnp.dot(ha, w2a_ref[0], preferred_element_type=jnp.float32)
        o = o + jnp.dot(hb, w2b_ref[0], preferred_element_type=jnp.float32)
        o_ref[...] = o + b2_ref[0, 0]


def kernel(x, Wr, W1, b1, W2, b2):
    Bx, Tx, Cx = x.shape
    E, C, H = W1.shape
    N = Bx * Tx
    NB = N // _BT + E
    PADDED = NB * _BT

    x_flat = x.reshape(N, Cx)

    logits = x_flat @ Wr
    eid = jnp.argmax(logits, axis=1).astype(jnp.int32)

    G = 8
    NG = N // G
    eidg = eid.reshape(G, NG)
    ohg = (eidg[:, :, None] == jnp.arange(E, dtype=jnp.int32)).astype(jnp.int32)
    ohp = ohg.transpose(1, 0, 2)
    cs = jnp.cumsum(ohp.reshape(NG, G * E), axis=0)
    totals = cs[-1].reshape(G, E)
    goff = jnp.concatenate(
        [jnp.zeros((1, E), jnp.int32), jnp.cumsum(totals, axis=0)[:-1]], axis=0)
    counts = jnp.sum(totals, axis=0)
    rank_incl = cs.reshape(NG, G, E) + goff[None]
    rank = jnp.sum(ohp * rank_incl, axis=2)
    rank = rank.T.reshape(N) - 1
    padded = ((counts + _BT - 1) // _BT) * _BT
    bounds = jnp.cumsum(padded)
    starts = bounds - padded
    pos = (starts[eid] + rank).astype(jnp.int32)
    total = bounds[-1]
    nb_active = (total // _BT).astype(jnp.int32)

    bidx = jnp.arange(NB, dtype=jnp.int32) * _BT
    block_expert = jnp.sum((bounds[None, :] <= bidx[:, None]).astype(jnp.int32),
                           axis=1)
    block_expert = jnp.minimum(block_expert, E - 1)
    last_e = block_expert[jnp.maximum(nb_active - 1, 0)]
    block_expert = jnp.where(bidx < total, block_expert, last_e)

    x_sorted = _sc_row_scatter(x_flat, pos, PADDED)

    grid_spec = pltpu.PrefetchScalarGridSpec(
        num_scalar_prefetch=2,
        grid=(NB,),
        in_specs=[
            pl.BlockSpec((_BT, C), lambda b, be, act: (b, 0)),
            pl.BlockSpec((1, C, H // 2), lambda b, be, act: (be[b], 0, 0)),
            pl.BlockSpec((1, C, H // 2), lambda b, be, act: (be[b], 0, 1)),
            pl.BlockSpec((1, 1, H), lambda b, be, act: (be[b], 0, 0)),
            pl.BlockSpec((1, H // 2, C), lambda b, be, act: (be[b], 0, 0)),
            pl.BlockSpec((1, H // 2, C), lambda b, be, act: (be[b], 1, 0)),
            pl.BlockSpec((1, 1, C), lambda b, be, act: (be[b], 0, 0)),
        ],
        out_specs=pl.BlockSpec((_BT, C), lambda b, be, act: (b, 0)),
    )
    out_buf = pl.pallas_call(
        _ffn_block,
        grid_spec=grid_spec,
        out_shape=jax.ShapeDtypeStruct((PADDED, C), jnp.float32),
    )(block_expert, jnp.full((1,), nb_active, jnp.int32),
      x_sorted, W1, W1, b1.reshape(E, 1, H), W2, W2, b2.reshape(E, 1, C))

    y_flat = _sc_row_gather(out_buf, pos)
    return y_flat.reshape(Bx, Tx, Cx)
